# Initial kernel scaffold; baseline (speedup 1.0000x reference)
import jax
import jax.numpy as jnp
from jax.experimental import pallas as pl


def kernel(x, seg_ids, v, lin):
    raise NotImplementedError("write your pallas kernel here")


# speedup vs baseline: 43.5867x; 43.5867x over previous
import jax
import jax.numpy as jnp
from jax.experimental import pallas as pl

_SCALE = 100.0
_B = 128


def _ect_block_kernel(seg_ref, x_ref, vt_ref, lg_ref, out_ref):
    i = pl.program_id(0)
    bn = x_ref.shape[0]
    n_groups = lg_ref.shape[0]

    nh = jnp.dot(x_ref[...], vt_ref[...], preferred_element_type=jnp.float32)

    seg = seg_ref[0, 0, :].reshape(bn, 1)
    col = jax.lax.broadcasted_iota(jnp.int32, (bn, _B), 1)
    onehot = (seg == col).astype(jnp.float32)

    @pl.when(i == 0)
    def _init():
        out_ref[...] = jnp.zeros_like(out_ref)

    for g in range(n_groups):
        z = _SCALE * (lg_ref[g : g + 1, :] - nh)
        ecc = jax.nn.sigmoid(z)
        part = jax.lax.dot_general(
            onehot, ecc, (((0,), (0,)), ((), ())),
            preferred_element_type=jnp.float32,
        )
        out_ref[:, g * 128 : (g + 1) * 128] += part


def kernel(x, seg_ids, v, lin):
    n, _ = x.shape
    t = v.shape[1]
    lin_flat = lin.reshape(-1)
    s = lin_flat.shape[0]
    st = s * t
    assert st % 128 == 0 and 128 % t == 0
    s_per_group = 128 // t
    n_groups = st // 128

    bn = 2000
    g_steps = -(-n // bn)
    n_pad = g_steps * bn
    if n_pad != n:
        x = jnp.pad(x, ((0, n_pad - n), (0, 0)))
        seg_ids = jnp.pad(seg_ids, (0, n_pad - n), constant_values=_B)
    seg3 = seg_ids.astype(jnp.int32).reshape(g_steps, 1, bn)

    vt = jnp.concatenate([v] * s_per_group, axis=1)
    lane = jnp.arange(128) // t
    lg = lin_flat[s_per_group * jnp.arange(n_groups)[:, None] + lane[None, :]]

    out = pl.pallas_call(
        _ect_block_kernel,
        grid=(g_steps,),
        in_specs=[
            pl.BlockSpec((1, 1, bn), lambda i: (i, 0, 0)),
            pl.BlockSpec((bn, x.shape[1]), lambda i: (i, 0)),
            pl.BlockSpec(vt.shape, lambda i: (0, 0)),
            pl.BlockSpec(lg.shape, lambda i: (0, 0)),
        ],
        out_specs=pl.BlockSpec((_B, st), lambda i: (0, 0)),
        out_shape=jax.ShapeDtypeStruct((_B, st), jnp.float32),
    )(seg3, x, vt, lg)

    return out.reshape(_B, 1, s, t)
